# Initial kernel scaffold; baseline (speedup 1.0000x reference)
import jax
import jax.numpy as jnp
from jax.experimental import pallas as pl


def kernel(x, edge_index, batch, W_gcn, b_gcn, W1, b1, bn_gamma, bn_beta, W2, b2):
    raise NotImplementedError("write your pallas kernel here")


# speedup vs baseline: 174.0900x; 174.0900x over previous
import functools

import jax
import jax.numpy as jnp
from jax import lax
from jax.experimental import pallas as pl
from jax.experimental.pallas import tpu as pltpu
from jax.experimental.pallas import tpu_sc as plsc

N, E, D, G, OUT = 10000, 640000, 128, 64, 2
NP = 10240
NC, NS, L = 2, 16, 16
NW = NC * NS
NCH = 157
EPT = NCH * 128
EP = NW * EPT
NPT = NP // NS
BN_SCALE = 1.0 / (1.0 + 1e-5) ** 0.5

def _deg_body(dst_hbm, zeros_hbm, out_hbm, idx_v, ones_v, deg_sh):
    cid = lax.axis_index("c")
    sid = lax.axis_index("s")
    tile = cid * NS + sid

    @pl.when(sid == 0)
    def _():
        pltpu.sync_copy(zeros_hbm, deg_sh)

    for k in range(8):
        ones_v[pl.ds(k * L, L)] = jnp.full((L,), 1.0, jnp.float32)
    pltpu.sync_copy(dst_hbm.at[tile], idx_v)
    plsc.subcore_barrier()

    @pl.loop(0, NCH)
    def _(j):
        pltpu.sync_copy(ones_v, deg_sh.at[idx_v.at[j]], add=True)

    plsc.subcore_barrier()
    pltpu.sync_copy(deg_sh.at[pl.ds(sid * NPT, NPT)],
                    out_hbm.at[cid, pl.ds(sid * NPT, NPT)])


def _agg_body(src_hbm, dst_hbm, g_hbm, zeros_hbm, out_hbm,
              idxs_v, idxd_v, vals_v, g_loc, g_sh, agg_sh):
    cid = lax.axis_index("c")
    sid = lax.axis_index("s")
    tile = cid * NS + sid

    @pl.when(sid == 0)
    def _():
        pltpu.sync_copy(zeros_hbm, agg_sh)
        pltpu.sync_copy(g_hbm, g_sh)

    pltpu.sync_copy(src_hbm.at[tile], idxs_v)
    pltpu.sync_copy(dst_hbm.at[tile], idxd_v)
    plsc.subcore_barrier()
    pltpu.sync_copy(g_sh, g_loc)

    @pl.loop(0, NCH)
    def _(j):
        for k in range(8):
            ii = idxs_v[j, pl.ds(k * L, L)]
            vals_v[j, pl.ds(k * L, L)] = plsc.load_gather(g_loc, [ii])
        pltpu.sync_copy(vals_v.at[j], agg_sh.at[idxd_v.at[j]], add=True)

    plsc.subcore_barrier()
    pltpu.sync_copy(agg_sh.at[pl.ds(sid * NPT, NPT)],
                    out_hbm.at[cid, pl.ds(sid * NPT, NPT)])


@functools.lru_cache(maxsize=1)
def _sc_kernels():
    mesh = plsc.VectorSubcoreMesh(core_axis_name="c", subcore_axis_name="s",
                                  num_cores=NC, num_subcores=NS)
    params = pltpu.CompilerParams(needs_layout_passes=False)
    deg_kernel = pl.kernel(
        _deg_body,
        compiler_params=params,
        out_type=jax.ShapeDtypeStruct((NC, NP), jnp.float32),
        mesh=mesh,
        scratch_types=[
            pltpu.VMEM((NCH, 128), jnp.int32),
            pltpu.VMEM((128,), jnp.float32),
            pltpu.VMEM_SHARED((NP,), jnp.float32),
        ],
    )
    agg_kernel = pl.kernel(
        _agg_body,
        compiler_params=params,
        out_type=jax.ShapeDtypeStruct((NC, NP), jnp.float32),
        mesh=mesh,
        scratch_types=[
            pltpu.VMEM((NCH, 128), jnp.int32),
            pltpu.VMEM((NCH, 128), jnp.int32),
            pltpu.VMEM((NCH, 128), jnp.float32),
            pltpu.VMEM((NP,), jnp.float32),
            pltpu.VMEM_SHARED((NP,), jnp.float32),
            pltpu.VMEM_SHARED((NP,), jnp.float32),
        ],
    )
    return deg_kernel, agg_kernel


def _c_body(x_ref, w_ref, degp_ref, g_ref, dinv_ref):
    h = jnp.dot(x_ref[...], w_ref[...], preferred_element_type=jnp.float32)[:, 0]
    deg = degp_ref[0, :] + degp_ref[1, :] + 1.0
    dinv = lax.rsqrt(deg)
    g_ref[...] = h * dinv
    dinv_ref[...] = dinv


_c_call = pl.pallas_call(
    _c_body,
    grid=(5,),
    in_specs=[
        pl.BlockSpec((2048, D), lambda i: (i, 0)),
        pl.BlockSpec((D, 1), lambda i: (0, 0)),
        pl.BlockSpec((2, 2048), lambda i: (0, i)),
    ],
    out_specs=[
        pl.BlockSpec((2048,), lambda i: (i,)),
        pl.BlockSpec((2048,), lambda i: (i,)),
    ],
    out_shape=[
        jax.ShapeDtypeStruct((NP,), jnp.float32),
        jax.ShapeDtypeStruct((NP,), jnp.float32),
    ],
)


def _e_body(aggp_ref, g_ref, dinv_ref, batch_ref, bgcn_ref, w1_ref, b1_ref,
            gam_ref, bet_ref, w2_ref, b2_ref, out_ref):
    s = aggp_ref[0, :] + aggp_ref[1, :]
    z = jnp.maximum(dinv_ref[...] * (s + g_ref[...]) + bgcn_ref[0], 0.0)
    grp = lax.broadcasted_iota(jnp.int32, (NP, G), 1)
    m = (batch_ref[...][:, None] == grp).astype(jnp.float32)
    sums = jnp.dot(z[None, :], m, preferred_element_type=jnp.float32)[0]
    counts = jnp.sum(m, axis=0)
    pooled = sums / jnp.maximum(counts, 1.0)
    t = pooled * w1_ref[0, 0] + b1_ref[0]
    t = t * (gam_ref[0] * BN_SCALE) + bet_ref[0]
    t = jnp.maximum(t, 0.0)
    o = t[:, None] * w2_ref[...] + b2_ref[...][None, :]
    out_ref[...] = jax.nn.sigmoid(o)


_e_call = pl.pallas_call(
    _e_body,
    out_shape=jax.ShapeDtypeStruct((G, OUT), jnp.float32),
)


def kernel(x, edge_index, batch, W_gcn, b_gcn, W1, b1, bn_gamma, bn_beta, W2, b2):
    src = edge_index[0]
    dst = edge_index[1]
    pad = EP - E
    src3 = jnp.concatenate([src, jnp.zeros((pad,), jnp.int32)]).reshape(NW, NCH, 128)
    dst3 = jnp.concatenate([dst, jnp.full((pad,), N, jnp.int32)]).reshape(NW, NCH, 128)
    x_pad = jnp.concatenate([x, jnp.zeros((NP - N, D), jnp.float32)], axis=0)
    batch_pad = jnp.concatenate([batch, jnp.full((NP - N,), G + 63, jnp.int32)])
    zeros_np = jnp.zeros((NP,), jnp.float32)
    deg_kernel, agg_kernel = _sc_kernels()
    degp = deg_kernel(dst3, zeros_np)
    g, dinv = _c_call(x_pad, W_gcn, degp)
    aggp = agg_kernel(src3, dst3, g, zeros_np)
    return _e_call(aggp, g, dinv, batch_pad, b_gcn, W1, b1, bn_gamma, bn_beta, W2, b2)


# speedup vs baseline: 192.6134x; 1.1064x over previous
import functools

import jax
import jax.numpy as jnp
from jax import lax
from jax.experimental import pallas as pl
from jax.experimental.pallas import tpu as pltpu
from jax.experimental.pallas import tpu_sc as plsc

N, E, D, G, OUT = 10000, 640000, 128, 64, 2
NP = 10240
NC, NS, L = 2, 16, 16
NW = NC * NS
NCH = 157
EPT = NCH * 128
EP = NW * EPT
NPT = NP // NS
BN_SCALE = 1.0 / (1.0 + 1e-5) ** 0.5

def _deg_body(dst_hbm, zeros_hbm, out_hbm, idx_v, ones_v, deg_sh):
    cid = lax.axis_index("c")
    sid = lax.axis_index("s")
    tile = cid * NS + sid

    @pl.when(sid == 0)
    def _():
        pltpu.sync_copy(zeros_hbm, deg_sh)

    @pl.loop(0, EPT // L)
    def _(j):
        ones_v[pl.ds(j * L, L)] = jnp.full((L,), 1.0, jnp.float32)

    pltpu.sync_copy(dst_hbm.at[tile], idx_v)
    plsc.subcore_barrier()
    pltpu.sync_copy(ones_v, deg_sh.at[idx_v], add=True)
    plsc.subcore_barrier()
    pltpu.sync_copy(deg_sh.at[pl.ds(sid * NPT, NPT)],
                    out_hbm.at[cid, pl.ds(sid * NPT, NPT)])


def _agg_body(src_hbm, dst_hbm, g_hbm, zeros_hbm, out_hbm,
              idxs_v, idxd_v, vals_v, g_loc, g_sh, agg_sh):
    cid = lax.axis_index("c")
    sid = lax.axis_index("s")
    tile = cid * NS + sid

    @pl.when(sid == 0)
    def _():
        pltpu.sync_copy(zeros_hbm, agg_sh)
        pltpu.sync_copy(g_hbm, g_sh)

    pltpu.sync_copy(src_hbm.at[tile], idxs_v)
    pltpu.sync_copy(dst_hbm.at[tile], idxd_v)
    plsc.subcore_barrier()
    pltpu.sync_copy(g_sh, g_loc)

    @pl.loop(0, EPT // L)
    def _(j):
        ii = idxs_v[pl.ds(j * L, L)]
        vals_v[pl.ds(j * L, L)] = plsc.load_gather(g_loc, [ii])

    pltpu.sync_copy(vals_v, agg_sh.at[idxd_v], add=True)
    plsc.subcore_barrier()
    pltpu.sync_copy(agg_sh.at[pl.ds(sid * NPT, NPT)],
                    out_hbm.at[cid, pl.ds(sid * NPT, NPT)])


@functools.lru_cache(maxsize=1)
def _sc_kernels():
    mesh = plsc.VectorSubcoreMesh(core_axis_name="c", subcore_axis_name="s",
                                  num_cores=NC, num_subcores=NS)
    params = pltpu.CompilerParams(needs_layout_passes=False)
    deg_kernel = pl.kernel(
        _deg_body,
        compiler_params=params,
        out_type=jax.ShapeDtypeStruct((NC, NP), jnp.float32),
        mesh=mesh,
        scratch_types=[
            pltpu.VMEM((EPT,), jnp.int32),
            pltpu.VMEM((EPT,), jnp.float32),
            pltpu.VMEM_SHARED((NP,), jnp.float32),
        ],
    )
    agg_kernel = pl.kernel(
        _agg_body,
        compiler_params=params,
        out_type=jax.ShapeDtypeStruct((NC, NP), jnp.float32),
        mesh=mesh,
        scratch_types=[
            pltpu.VMEM((EPT,), jnp.int32),
            pltpu.VMEM((EPT,), jnp.int32),
            pltpu.VMEM((EPT,), jnp.float32),
            pltpu.VMEM((NP,), jnp.float32),
            pltpu.VMEM_SHARED((NP,), jnp.float32),
            pltpu.VMEM_SHARED((NP,), jnp.float32),
        ],
    )
    return deg_kernel, agg_kernel


def _c_body(x_ref, w_ref, degp_ref, g_ref, dinv_ref):
    h = jnp.dot(x_ref[...], w_ref[...], preferred_element_type=jnp.float32)[:, 0]
    deg = degp_ref[0, :] + degp_ref[1, :] + 1.0
    dinv = lax.rsqrt(deg)
    g_ref[...] = h * dinv
    dinv_ref[...] = dinv


_c_call = pl.pallas_call(
    _c_body,
    grid=(5,),
    in_specs=[
        pl.BlockSpec((2048, D), lambda i: (i, 0)),
        pl.BlockSpec((D, 1), lambda i: (0, 0)),
        pl.BlockSpec((2, 2048), lambda i: (0, i)),
    ],
    out_specs=[
        pl.BlockSpec((2048,), lambda i: (i,)),
        pl.BlockSpec((2048,), lambda i: (i,)),
    ],
    out_shape=[
        jax.ShapeDtypeStruct((NP,), jnp.float32),
        jax.ShapeDtypeStruct((NP,), jnp.float32),
    ],
)


def _e_body(aggp_ref, g_ref, dinv_ref, batch_ref, bgcn_ref, w1_ref, b1_ref,
            gam_ref, bet_ref, w2_ref, b2_ref, out_ref):
    s = aggp_ref[0, :] + aggp_ref[1, :]
    z = jnp.maximum(dinv_ref[...] * (s + g_ref[...]) + bgcn_ref[0], 0.0)
    grp = lax.broadcasted_iota(jnp.int32, (NP, G), 1)
    m = (batch_ref[...][:, None] == grp).astype(jnp.float32)
    sums = jnp.dot(z[None, :], m, preferred_element_type=jnp.float32)[0]
    counts = jnp.sum(m, axis=0)
    pooled = sums / jnp.maximum(counts, 1.0)
    t = pooled * w1_ref[0, 0] + b1_ref[0]
    t = t * (gam_ref[0] * BN_SCALE) + bet_ref[0]
    t = jnp.maximum(t, 0.0)
    o = t[:, None] * w2_ref[...] + b2_ref[...][None, :]
    out_ref[...] = jax.nn.sigmoid(o)


_e_call = pl.pallas_call(
    _e_body,
    out_shape=jax.ShapeDtypeStruct((G, OUT), jnp.float32),
)


def kernel(x, edge_index, batch, W_gcn, b_gcn, W1, b1, bn_gamma, bn_beta, W2, b2):
    src = edge_index[0]
    dst = edge_index[1]
    pad = EP - E
    src3 = jnp.concatenate([src, jnp.zeros((pad,), jnp.int32)]).reshape(NW, EPT)
    dst3 = jnp.concatenate([dst, jnp.full((pad,), N, jnp.int32)]).reshape(NW, EPT)
    x_pad = jnp.concatenate([x, jnp.zeros((NP - N, D), jnp.float32)], axis=0)
    batch_pad = jnp.concatenate([batch, jnp.full((NP - N,), G + 63, jnp.int32)])
    zeros_np = jnp.zeros((NP,), jnp.float32)
    deg_kernel, agg_kernel = _sc_kernels()
    degp = deg_kernel(dst3, zeros_np)
    g, dinv = _c_call(x_pad, W_gcn, degp)
    aggp = agg_kernel(src3, dst3, g, zeros_np)
    return _e_call(aggp, g, dinv, batch_pad, b_gcn, W1, b1, bn_gamma, bn_beta, W2, b2)


# speedup vs baseline: 240.2019x; 1.2471x over previous
import functools

import jax
import jax.numpy as jnp
from jax import lax
from jax.experimental import pallas as pl
from jax.experimental.pallas import tpu as pltpu
from jax.experimental.pallas import tpu_sc as plsc

N, E, D, G, OUT = 10000, 640000, 128, 64, 2
NP = 10240
NC, NS, L = 2, 16, 16
NW = NC * NS
EPT = E // NW
NPT = NP // NS
XB = 2048
BN_SCALE = 1.0 / (1.0 + 1e-5) ** 0.5


def _rsqrt16(d):
    i = plsc.bitcast(d, jnp.int32)
    i = jnp.full((L,), 0x5F3759DF, jnp.int32) - (i >> 1)
    y = plsc.bitcast(i, jnp.float32)
    for _ in range(3):
        y = y * (1.5 - 0.5 * d * y * y)
    return y


def _deg_body(ei_hbm, zeros_hbm, out_hbm, idx_v, ones_v, deg_sh):
    cid = lax.axis_index("c")
    sid = lax.axis_index("s")
    tile = cid * NS + sid

    @pl.when(sid == 0)
    def _():
        pltpu.sync_copy(zeros_hbm, deg_sh)

    @pl.loop(0, EPT // L)
    def _(j):
        ones_v[pl.ds(j * L, L)] = jnp.full((L,), 1.0, jnp.float32)

    pltpu.sync_copy(ei_hbm.at[pl.ds(E + tile * EPT, EPT)], idx_v)
    plsc.subcore_barrier()
    pltpu.sync_copy(ones_v, deg_sh.at[idx_v], add=True)
    plsc.subcore_barrier()
    pltpu.sync_copy(deg_sh.at[pl.ds(sid * NPT, NPT)],
                    out_hbm.at[cid, pl.ds(sid * NPT, NPT)])


def _agg_body(ei_hbm, h_hbm, degp_hbm, zeros_hbm, out_hbm,
              idxs_v, idxd_v, vals_v, g_loc, d0_v, d1_v, h_v, g_v,
              g_sh, agg_sh):
    cid = lax.axis_index("c")
    sid = lax.axis_index("s")
    tile = cid * NS + sid
    nbase = sid * NPT

    @pl.when(sid == 0)
    def _():
        pltpu.sync_copy(zeros_hbm, agg_sh)

    pltpu.sync_copy(degp_hbm.at[0, pl.ds(nbase, NPT)], d0_v)
    pltpu.sync_copy(degp_hbm.at[1, pl.ds(nbase, NPT)], d1_v)
    pltpu.sync_copy(h_hbm.at[pl.ds(nbase, NPT)], h_v)

    @pl.loop(0, NPT // L)
    def _(j):
        sl = pl.ds(j * L, L)
        d = d0_v[sl] + d1_v[sl] + 1.0
        g_v[sl] = h_v[sl] * _rsqrt16(d)

    pltpu.sync_copy(g_v, g_sh.at[pl.ds(nbase, NPT)])
    pltpu.sync_copy(ei_hbm.at[pl.ds(tile * EPT, EPT)], idxs_v)
    pltpu.sync_copy(ei_hbm.at[pl.ds(E + tile * EPT, EPT)], idxd_v)
    plsc.subcore_barrier()
    pltpu.sync_copy(g_sh, g_loc)

    @pl.loop(0, EPT // L)
    def _(j):
        sl = pl.ds(j * L, L)
        vals_v[sl] = plsc.load_gather(g_loc, [idxs_v[sl]])

    pltpu.sync_copy(vals_v, agg_sh.at[idxd_v], add=True)
    plsc.subcore_barrier()
    pltpu.sync_copy(agg_sh.at[pl.ds(nbase, NPT)],
                    out_hbm.at[cid, pl.ds(nbase, NPT)])


@functools.lru_cache(maxsize=1)
def _sc_kernels():
    mesh = plsc.VectorSubcoreMesh(core_axis_name="c", subcore_axis_name="s",
                                  num_cores=NC, num_subcores=NS)
    params = pltpu.CompilerParams(needs_layout_passes=False)
    deg_kernel = pl.kernel(
        _deg_body,
        compiler_params=params,
        out_type=jax.ShapeDtypeStruct((NC, NP), jnp.float32),
        mesh=mesh,
        scratch_types=[
            pltpu.VMEM((EPT,), jnp.int32),
            pltpu.VMEM((EPT,), jnp.float32),
            pltpu.VMEM_SHARED((NP,), jnp.float32),
        ],
    )
    agg_kernel = pl.kernel(
        _agg_body,
        compiler_params=params,
        out_type=jax.ShapeDtypeStruct((NC, NP), jnp.float32),
        mesh=mesh,
        scratch_types=[
            pltpu.VMEM((EPT,), jnp.int32),
            pltpu.VMEM((EPT,), jnp.int32),
            pltpu.VMEM((EPT,), jnp.float32),
            pltpu.VMEM((NP,), jnp.float32),
            pltpu.VMEM((NPT,), jnp.float32),
            pltpu.VMEM((NPT,), jnp.float32),
            pltpu.VMEM((NPT,), jnp.float32),
            pltpu.VMEM((NPT,), jnp.float32),
            pltpu.VMEM_SHARED((NP,), jnp.float32),
            pltpu.VMEM_SHARED((NP,), jnp.float32),
        ],
    )
    return deg_kernel, agg_kernel


def _c1_body(x_ref, w_ref, h_ref):
    i = pl.program_id(0)
    h = jnp.dot(x_ref[...], w_ref[...], preferred_element_type=jnp.float32)[:, 0]
    row = i * XB + lax.broadcasted_iota(jnp.int32, (XB,), 0)
    h_ref[...] = jnp.where(row < N, h, 0.0)


_c1_call = pl.pallas_call(
    _c1_body,
    grid=(NP // XB,),
    in_specs=[
        pl.BlockSpec((XB, D), lambda i: (i, 0)),
        pl.BlockSpec((D, 1), lambda i: (0, 0)),
    ],
    out_specs=pl.BlockSpec((XB,), lambda i: (i,)),
    out_shape=jax.ShapeDtypeStruct((NP,), jnp.float32),
)


def _e_body(degp_ref, h_ref, aggp_ref, batch_ref, bgcn_ref, w1_ref, b1_ref,
            gam_ref, bet_ref, w2_ref, b2_ref, out_ref):
    deg = degp_ref[0, :] + degp_ref[1, :] + 1.0
    dinv = lax.rsqrt(deg)
    g = h_ref[...] * dinv
    s = aggp_ref[0, :] + aggp_ref[1, :]
    z = jnp.maximum(dinv * (s + g) + bgcn_ref[0], 0.0)
    grp = lax.broadcasted_iota(jnp.int32, (NP, G), 1)
    m = (batch_ref[...][:, None] == grp).astype(jnp.float32)
    sums = jnp.dot(z[None, :], m, preferred_element_type=jnp.float32)[0]
    counts = jnp.sum(m, axis=0)
    pooled = sums / jnp.maximum(counts, 1.0)
    t = pooled * w1_ref[0, 0] + b1_ref[0]
    t = t * (gam_ref[0] * BN_SCALE) + bet_ref[0]
    t = jnp.maximum(t, 0.0)
    o = t[:, None] * w2_ref[...] + b2_ref[...][None, :]
    out_ref[...] = jax.nn.sigmoid(o)


_e_call = pl.pallas_call(
    _e_body,
    out_shape=jax.ShapeDtypeStruct((G, OUT), jnp.float32),
)


def kernel(x, edge_index, batch, W_gcn, b_gcn, W1, b1, bn_gamma, bn_beta, W2, b2):
    batch_pad = jnp.concatenate([batch, jnp.full((NP - N,), G + 63, jnp.int32)])
    zeros_np = jnp.zeros((NP,), jnp.float32)
    deg_kernel, agg_kernel = _sc_kernels()
    h = _c1_call(x, W_gcn)
    ei_flat = edge_index.reshape(2 * E)
    degp = deg_kernel(ei_flat, zeros_np)
    aggp = agg_kernel(ei_flat, h, degp, zeros_np)
    return _e_call(degp, h, aggp, batch_pad, b_gcn, W1, b1, bn_gamma,
                   bn_beta, W2, b2)
